# baseline (speedup 1.0000x reference)
import functools
import math

import jax
import jax.numpy as jnp
from jax.experimental import pallas as pl
from jax.experimental.pallas import tpu as pltpu


def _fused_kernel(x_ref, w_ref, b_ref, g_ref, beta_ref, out_ref, ypre_ref,
                  *, n_topk, n_batch):
    C = x_ref.shape[1]
    N = x_ref.shape[2]
    step = pl.program_id(0)

    @pl.when(step < n_batch)
    def _graph_conv():
        A = x_ref[pl.ds(jnp.minimum(step, n_batch - 1), 1)][0]
        nrm = jnp.sqrt(jnp.sum(A * A, axis=0, keepdims=True))
        An = A / jnp.maximum(nrm, 1e-12)
        cs = jax.lax.dot_general(
            An, An, (((0,), (0,)), ((), ())),
            preferred_element_type=jnp.float32)

        neg = jnp.float32(-jnp.inf)
        weights = []
        onehots = []
        for j in range(n_topk + 1):
            m = jnp.max(cs, axis=0, keepdims=True)
            eq = cs == m
            cs = jnp.where(eq, neg, cs)
            if j > 0:
                weights.append(m)
                onehots.append(eq)

        exps = [jnp.exp(w - weights[0]) for w in weights]
        denom = exps[0]
        for e in exps[1:]:
            denom = denom + e
        wsm = [e / denom for e in exps]

        Ab = A.astype(jnp.bfloat16)
        parts = []
        for k in range(n_topk):
            gath = jax.lax.dot_general(
                Ab, onehots[k].astype(jnp.bfloat16),
                (((1,), (0,)), ((), ())),
                preferred_element_type=jnp.float32)
            parts.append((gath * wsm[k]).astype(jnp.bfloat16))
        stack = jnp.concatenate(parts, axis=0)
        ypre_ref[pl.ds(jnp.minimum(step, n_batch - 1), 1)] = jax.lax.dot_general(
            w_ref[...].astype(jnp.bfloat16), stack,
            (((1,), (0,)), ((), ())),
            preferred_element_type=jnp.float32)[None]

    @pl.when(step == n_batch)
    def _bn_gelu():
        cnt = jnp.float32(n_batch * N)
        s = jnp.zeros((C, 1), dtype=jnp.float32)
        ss = jnp.zeros((C, 1), dtype=jnp.float32)
        for b in range(n_batch):
            yb = ypre_ref[b] + b_ref[...]
            s = s + jnp.sum(yb, axis=1, keepdims=True)
            ss = ss + jnp.sum(yb * yb, axis=1, keepdims=True)
        mean = s / cnt
        var = ss / cnt - mean * mean
        inv = jax.lax.rsqrt(var + 1e-5) * g_ref[...]
        shift = beta_ref[...] - mean * inv
        inv_sqrt2 = jnp.float32(1.0 / math.sqrt(2.0))
        for b in range(n_batch):
            t = (ypre_ref[b] + b_ref[...]) * inv + shift
            gel = 0.5 * t * (1.0 + jax.lax.erf(t * inv_sqrt2))
            out_ref[b] = gel + x_ref[b]


def kernel(x, W_conv, b_conv, gamma, beta):
    B, C, H, W = x.shape
    N = H * W
    topk = W_conv.shape[1] // C
    x3 = x.reshape(B, C, N)

    out = pl.pallas_call(
        functools.partial(_fused_kernel, n_topk=topk, n_batch=B),
        grid=(B + 1,),
        in_specs=[
            pl.BlockSpec((B, C, N), lambda i: (0, 0, 0)),
            pl.BlockSpec((C, topk * C), lambda i: (0, 0)),
            pl.BlockSpec((C, 1), lambda i: (0, 0)),
            pl.BlockSpec((C, 1), lambda i: (0, 0)),
            pl.BlockSpec((C, 1), lambda i: (0, 0)),
        ],
        out_specs=pl.BlockSpec((B, C, N), lambda i: (0, 0, 0)),
        out_shape=jax.ShapeDtypeStruct((B, C, N), jnp.float32),
        scratch_shapes=[pltpu.VMEM((B, C, N), jnp.float32)],
    )(x3, W_conv, b_conv.reshape(C, 1), gamma.reshape(C, 1),
      beta.reshape(C, 1))

    return out.reshape(B, C, H, W)


# speedup vs baseline: 1.0262x; 1.0262x over previous
import functools
import math

import jax
import jax.numpy as jnp
from jax.experimental import pallas as pl
from jax.experimental.pallas import tpu as pltpu


def _fused_kernel(x_ref, w_ref, eye_ref, g_ref, beta_ref, out_ref, ypre_ref,
                  *, n_topk, n_batch):
    C = x_ref.shape[1]
    N = x_ref.shape[2]
    step = pl.program_id(0)

    @pl.when(step < n_batch)
    def _graph_conv():
        A = x_ref[pl.ds(jnp.minimum(step, n_batch - 1), 1)][0]
        nrm = jnp.sqrt(jnp.sum(A * A, axis=0, keepdims=True))
        inv = 1.0 / jnp.maximum(nrm, 1e-12)
        An = A * inv
        cs = jax.lax.dot_general(
            An, A, (((0,), (0,)), ((), ())),
            preferred_element_type=jnp.float32)
        cs = cs + eye_ref[...]

        weights = []
        onehots = []
        neg = jnp.float32(-jnp.inf)
        for j in range(n_topk):
            m = jnp.max(cs, axis=0, keepdims=True)
            eq = cs == m
            if j + 1 < n_topk:
                cs = jnp.where(eq, neg, cs)
            weights.append(m * inv)
            onehots.append(eq)

        exps = [jnp.exp(w - weights[0]) for w in weights]
        denom = exps[0]
        for e in exps[1:]:
            denom = denom + e
        wsm = [e / denom for e in exps]

        Ab = A.astype(jnp.bfloat16)
        parts = []
        for k in range(n_topk):
            gath = jax.lax.dot_general(
                Ab, onehots[k].astype(jnp.bfloat16),
                (((1,), (0,)), ((), ())),
                preferred_element_type=jnp.float32)
            parts.append((gath * wsm[k]).astype(jnp.bfloat16))
        stack = jnp.concatenate(parts, axis=0)
        ypre_ref[pl.ds(jnp.minimum(step, n_batch - 1), 1)] = jax.lax.dot_general(
            w_ref[...].astype(jnp.bfloat16), stack,
            (((1,), (0,)), ((), ())),
            preferred_element_type=jnp.float32)[None]

    @pl.when(step == n_batch)
    def _bn_gelu():
        cnt = jnp.float32(n_batch * N)
        s = jnp.zeros((C, 1), dtype=jnp.float32)
        ss = jnp.zeros((C, 1), dtype=jnp.float32)
        for b in range(n_batch):
            yb = ypre_ref[b]
            s = s + jnp.sum(yb, axis=1, keepdims=True)
            ss = ss + jnp.sum(yb * yb, axis=1, keepdims=True)
        mean = s / cnt
        var = ss / cnt - mean * mean
        inv = jax.lax.rsqrt(var + 1e-5) * g_ref[...]
        shift = beta_ref[...] - mean * inv
        inv_sqrt2 = jnp.float32(1.0 / math.sqrt(2.0))
        for b in range(n_batch):
            t = ypre_ref[b] * inv + shift
            gel = 0.5 * t * (1.0 + jax.lax.erf(t * inv_sqrt2))
            out_ref[b] = gel + x_ref[b]


def kernel(x, W_conv, b_conv, gamma, beta):
    B, C, H, W = x.shape
    N = H * W
    topk = W_conv.shape[1] // C
    x3 = x.reshape(B, C, N)
    eye_neg = (-1e30) * jnp.eye(N, dtype=jnp.float32)
    del b_conv

    out = pl.pallas_call(
        functools.partial(_fused_kernel, n_topk=topk, n_batch=B),
        grid=(B + 1,),
        in_specs=[
            pl.BlockSpec((B, C, N), lambda i: (0, 0, 0)),
            pl.BlockSpec((C, topk * C), lambda i: (0, 0)),
            pl.BlockSpec((N, N), lambda i: (0, 0)),
            pl.BlockSpec((C, 1), lambda i: (0, 0)),
            pl.BlockSpec((C, 1), lambda i: (0, 0)),
        ],
        out_specs=pl.BlockSpec((B, C, N), lambda i: (0, 0, 0)),
        out_shape=jax.ShapeDtypeStruct((B, C, N), jnp.float32),
        scratch_shapes=[pltpu.VMEM((B, C, N), jnp.float32)],
    )(x3, W_conv, eye_neg, gamma.reshape(C, 1), beta.reshape(C, 1))

    return out.reshape(B, C, H, W)
